# baseline (speedup 1.0000x reference)
import functools

import jax
import jax.numpy as jnp
from jax import lax
from jax.experimental import pallas as pl
from jax.experimental.pallas import tpu as pltpu
from jax.experimental.pallas import tpu_sc as plsc

N_NODES = 10000
NP = 10240
NE = 320000
NC = 2
NS = 16
NW = NC * NS
CB = 128
TCH = (NE + CB - 1) // CB + 60
K0 = 56
K1 = 104
CH = TCH // NW
EP = TCH * CB
RPT = NP // NS
ZR = 32

assert NS * (K0 + K1) == TCH

_MESH = plsc.VectorSubcoreMesh(core_axis_name="c", subcore_axis_name="s")


def _fill_const(ref, n16, val):
  v = jnp.full((16,), val, jnp.float32)

  @pl.loop(0, n16)
  def _(t):
    if len(ref.shape) == 1:
      ref[pl.ds(t * 16, 16)] = v
    else:
      w16 = ref.shape[1] // 16
      i = t // w16
      j = t % w16
      ref[i, pl.ds(j * 16, 16)] = v


def _deg_body(srcd_hbm, dstd_hbm, dego_hbm, degi_hbm,
              srcv, dstv, ones_v, zb, dego, degi):
  c = lax.axis_index("c")
  s = lax.axis_index("s")
  wid = c * NS + s
  row0 = s * RPT

  _fill_const(ones_v, CB // 16, 1.0)
  _fill_const(zb, RPT // 16, 0.0)
  pltpu.sync_copy(zb, dego.at[pl.ds(row0, RPT)])
  pltpu.sync_copy(zb, degi.at[pl.ds(row0, RPT)])
  pltpu.sync_copy(srcd_hbm.at[pl.ds(wid * CH, CH)], srcv)
  pltpu.sync_copy(dstd_hbm.at[pl.ds(wid * CH, CH)], dstv)
  plsc.subcore_barrier()

  @pl.loop(0, CH)
  def _(k):
    pltpu.sync_copy(ones_v, dego.at[srcv.at[k]], add=True)
    pltpu.sync_copy(ones_v, degi.at[dstv.at[k]], add=True)

  plsc.subcore_barrier()
  pltpu.sync_copy(dego.at[pl.ds(row0, RPT)], dego_hbm.at[c, pl.ds(row0, RPT)])
  pltpu.sync_copy(degi.at[pl.ds(row0, RPT)], degi_hbm.at[c, pl.ds(row0, RPT)])


_deg_kernel = pl.kernel(
    _deg_body,
    out_type=(
        jax.ShapeDtypeStruct((NC, NP), jnp.float32),
        jax.ShapeDtypeStruct((NC, NP), jnp.float32),
    ),
    mesh=_MESH,
    scratch_types=(
        pltpu.VMEM((CH, CB), jnp.int32),
        pltpu.VMEM((CH, CB), jnp.int32),
        pltpu.VMEM((CB,), jnp.float32),
        pltpu.VMEM((RPT,), jnp.float32),
        pltpu.VMEM_SHARED((NP,), jnp.float32),
        pltpu.VMEM_SHARED((NP,), jnp.float32),
    ),
)


def _agg_body(xs_hbm, src_hbm, dst_hbm, out_hbm, srcv, dstv, rows0, zb, acc):
  c = lax.axis_index("c")
  s = lax.axis_index("s")
  row0 = s * RPT
  d = xs_hbm.shape[1]

  _fill_const(zb, ZR * d // 16, 0.0)

  @pl.loop(0, RPT // ZR)
  def _(j):
    pltpu.sync_copy(zb, acc.at[pl.ds(row0 + j * ZR, ZR)])

  plsc.subcore_barrier()

  def run(base, nk):
    pltpu.sync_copy(src_hbm.at[pl.ds(base, nk)], srcv.at[pl.ds(0, nk)])
    pltpu.sync_copy(dst_hbm.at[pl.ds(base, nk)], dstv.at[pl.ds(0, nk)])

    @pl.loop(0, nk)
    def _(k):
      pltpu.sync_copy(xs_hbm.at[srcv.at[k]], rows0)
      pltpu.sync_copy(rows0, acc.at[dstv.at[k]], add=True)

  @pl.when(c == 0)
  def _():
    run(s * K0, K0)

  @pl.when(c == 1)
  def _():
    run(NS * K0 + s * K1, K1)

  plsc.subcore_barrier()
  pltpu.sync_copy(acc.at[pl.ds(row0, RPT)], out_hbm.at[c, pl.ds(row0, RPT)])


@functools.cache
def _agg_kernel(d):
  return pl.kernel(
      _agg_body,
      out_type=jax.ShapeDtypeStruct((NC, NP, d), jnp.float32),
      mesh=_MESH,
      compiler_params=pltpu.CompilerParams(use_tc_tiling_on_sc=False),
      scratch_types=(
          pltpu.VMEM((K1, CB), jnp.int32),
          pltpu.VMEM((K1, CB), jnp.int32),
          pltpu.VMEM((CB, d), jnp.float32),
          pltpu.VMEM((ZR, d), jnp.float32),
          pltpu.VMEM_SHARED((NP, d), jnp.float32),
      ),
  )


RB = 1000
_G = N_NODES // RB


def _rows(w):
  return pl.BlockSpec((RB, w), lambda i: (i, 0))


def _full(a, b):
  return pl.BlockSpec((a, b), lambda i: (0, 0))


def _dot(a, w):
  return jnp.dot(a, w, precision=lax.Precision.HIGHEST,
                 preferred_element_type=jnp.float32)


def _m0_body(x, do0, do1, di0, di1, xs1, ns, nd):
  nsv = lax.rsqrt(jnp.maximum(do0[...] + do1[...], 1.0))
  ndv = lax.rsqrt(jnp.maximum(di0[...] + di1[...], 1.0))
  ns[...] = nsv
  nd[...] = ndv
  xs1[...] = x[...] * nsv


def _m1_body(p0, p1, nd, ns, w1, b1, w2, xs2):
  a = (p0[...] + p1[...]) * nd[...]
  h = jnp.maximum(_dot(a, w1[...]) + b1[...], 0.0)
  xs2[...] = _dot(h * ns[...], w2[...])


def _m2_body(p0, p1, nd, ns, b2, w3, xs3):
  h = jnp.maximum((p0[...] + p1[...]) * nd[...] + b2[...], 0.0)
  xs3[...] = _dot(h * ns[...], w3[...])


def _m3_body(p0, p1, nd, ns, b3, xs4):
  xs4[...] = jnp.maximum((p0[...] + p1[...]) * nd[...] + b3[...], 0.0) * ns[...]


def _m4_body(p0, p1, nd, w4, b4, o):
  a = (p0[...] + p1[...]) * nd[...]
  o[...] = jnp.maximum(_dot(a, w4[...]) + b4[...], 0.0)


def _tc_call(body, in_specs, out_shapes, out_specs):
  return pl.pallas_call(
      body,
      grid=(_G,),
      in_specs=in_specs,
      out_specs=out_specs,
      out_shape=out_shapes,
  )


def kernel(x, edge_index, W1, b1, W2, b2, W3, b3, W4, b4):
  src = edge_index[0].astype(jnp.int32)
  dst = edge_index[1].astype(jnp.int32)
  pad = EP - NE
  pad0 = jnp.zeros((pad,), jnp.int32)
  padN = jnp.full((pad,), N_NODES, jnp.int32)
  src2 = jnp.concatenate([src, pad0]).reshape(TCH, CB)
  srcd2 = jnp.concatenate([src, padN]).reshape(TCH, CB)
  dst2 = jnp.concatenate([dst, padN]).reshape(TCH, CB)

  dego_p, degi_p = _deg_kernel(srcd2, dst2)
  do0 = dego_p[0, :N_NODES, None]
  do1 = dego_p[1, :N_NODES, None]
  di0 = degi_p[0, :N_NODES, None]
  di1 = degi_p[1, :N_NODES, None]

  xs1, ns, nd = _tc_call(
      _m0_body,
      [_rows(128), _rows(1), _rows(1), _rows(1), _rows(1)],
      (jax.ShapeDtypeStruct((N_NODES, 128), jnp.float32),
       jax.ShapeDtypeStruct((N_NODES, 1), jnp.float32),
       jax.ShapeDtypeStruct((N_NODES, 1), jnp.float32)),
      (_rows(128), _rows(1), _rows(1)),
  )(x, do0, do1, di0, di1)

  a1 = _agg_kernel(128)(xs1, src2, dst2)
  xs2 = _tc_call(
      _m1_body,
      [_rows(128), _rows(128), _rows(1), _rows(1),
       _full(128, 512), _full(1, 512), _full(512, 64)],
      jax.ShapeDtypeStruct((N_NODES, 64), jnp.float32),
      _rows(64),
  )(a1[0, :N_NODES], a1[1, :N_NODES], nd, ns, W1, b1[None, :], W2)

  a2 = _agg_kernel(64)(xs2, src2, dst2)
  xs3 = _tc_call(
      _m2_body,
      [_rows(64), _rows(64), _rows(1), _rows(1), _full(1, 64), _full(64, 16)],
      jax.ShapeDtypeStruct((N_NODES, 16), jnp.float32),
      _rows(16),
  )(a2[0, :N_NODES], a2[1, :N_NODES], nd, ns, b2[None, :], W3)

  a3 = _agg_kernel(16)(xs3, src2, dst2)
  xs4 = _tc_call(
      _m3_body,
      [_rows(16), _rows(16), _rows(1), _rows(1), _full(1, 16)],
      jax.ShapeDtypeStruct((N_NODES, 16), jnp.float32),
      _rows(16),
  )(a3[0, :N_NODES], a3[1, :N_NODES], nd, ns, b3[None, :])

  a4 = _agg_kernel(16)(xs4, src2, dst2)
  out = _tc_call(
      _m4_body,
      [_rows(16), _rows(16), _rows(1), _full(16, 128), _full(1, 128)],
      jax.ShapeDtypeStruct((N_NODES, 128), jnp.float32),
      _rows(128),
  )(a4[0, :N_NODES], a4[1, :N_NODES], nd, W4, b4[None, :])
  return out


# speedup vs baseline: 1.9207x; 1.9207x over previous
import functools

import jax
import jax.numpy as jnp
from jax import lax
from jax.experimental import pallas as pl
from jax.experimental.pallas import tpu as pltpu
from jax.experimental.pallas import tpu_sc as plsc

N_NODES = 10000
NP = 10240
NE = 320000
NC = 2
NS = 16
NW = NC * NS
CB = 128
TCH = (NE + CB - 1) // CB + 60
K0 = 80
K1 = 80
CH = TCH // NW
EP = TCH * CB
RPT = NP // NS
ZR = 32

assert NS * (K0 + K1) == TCH

_MESH = plsc.VectorSubcoreMesh(core_axis_name="c", subcore_axis_name="s")


def _fill_const(ref, n16, val):
  v = jnp.full((16,), val, jnp.float32)

  @pl.loop(0, n16)
  def _(t):
    if len(ref.shape) == 1:
      ref[pl.ds(t * 16, 16)] = v
    else:
      w16 = ref.shape[1] // 16
      i = t // w16
      j = t % w16
      ref[i, pl.ds(j * 16, 16)] = v


def _deg_body(srcd_hbm, dstd_hbm, dego_hbm, degi_hbm,
              srcv, dstv, ones_v, zb, dego, degi):
  c = lax.axis_index("c")
  s = lax.axis_index("s")
  wid = c * NS + s
  row0 = s * RPT

  _fill_const(ones_v, CB // 16, 1.0)
  _fill_const(zb, RPT // 16, 0.0)
  pltpu.sync_copy(zb, dego.at[pl.ds(row0, RPT)])
  pltpu.sync_copy(zb, degi.at[pl.ds(row0, RPT)])
  pltpu.sync_copy(srcd_hbm.at[pl.ds(wid * CH, CH)], srcv)
  pltpu.sync_copy(dstd_hbm.at[pl.ds(wid * CH, CH)], dstv)
  plsc.subcore_barrier()

  @pl.loop(0, CH)
  def _(k):
    pltpu.sync_copy(ones_v, dego.at[srcv.at[k]], add=True)
    pltpu.sync_copy(ones_v, degi.at[dstv.at[k]], add=True)

  plsc.subcore_barrier()
  pltpu.sync_copy(dego.at[pl.ds(row0, RPT)], dego_hbm.at[c, pl.ds(row0, RPT)])
  pltpu.sync_copy(degi.at[pl.ds(row0, RPT)], degi_hbm.at[c, pl.ds(row0, RPT)])


_deg_kernel = pl.kernel(
    _deg_body,
    out_type=(
        jax.ShapeDtypeStruct((NC, NP), jnp.float32),
        jax.ShapeDtypeStruct((NC, NP), jnp.float32),
    ),
    mesh=_MESH,
    scratch_types=(
        pltpu.VMEM((CH, CB), jnp.int32),
        pltpu.VMEM((CH, CB), jnp.int32),
        pltpu.VMEM((CB,), jnp.float32),
        pltpu.VMEM((RPT,), jnp.float32),
        pltpu.VMEM_SHARED((NP,), jnp.float32),
        pltpu.VMEM_SHARED((NP,), jnp.float32),
    ),
)


def _agg_body(xs_hbm, src_hbm, dst_hbm, out_hbm, srcv, dstv, rows0, zb, acc):
  c = lax.axis_index("c")
  s = lax.axis_index("s")
  row0 = s * RPT
  d = xs_hbm.shape[1]

  _fill_const(zb, ZR * d // 16, 0.0)

  @pl.loop(0, RPT // ZR)
  def _(j):
    pltpu.sync_copy(zb, acc.at[pl.ds(row0 + j * ZR, ZR)])

  plsc.subcore_barrier()

  def run(base, nk):
    pltpu.sync_copy(src_hbm.at[pl.ds(base, nk)], srcv.at[pl.ds(0, nk)])
    pltpu.sync_copy(dst_hbm.at[pl.ds(base, nk)], dstv.at[pl.ds(0, nk)])

    @pl.loop(0, nk)
    def _(k):
      pltpu.sync_copy(xs_hbm.at[srcv.at[k]], rows0)
      pltpu.sync_copy(rows0, acc.at[dstv.at[k]], add=True)

  @pl.when(c == 0)
  def _():
    run(s * K0, K0)

  @pl.when(c == 1)
  def _():
    run(NS * K0 + s * K1, K1)

  plsc.subcore_barrier()
  pltpu.sync_copy(acc.at[pl.ds(row0, RPT)], out_hbm.at[c, pl.ds(row0, RPT)])


@functools.cache
def _agg_kernel(d):
  return pl.kernel(
      _agg_body,
      out_type=jax.ShapeDtypeStruct((NC, NP, d), jnp.float32),
      mesh=_MESH,
      compiler_params=pltpu.CompilerParams(use_tc_tiling_on_sc=False),
      scratch_types=(
          pltpu.VMEM((K1, CB), jnp.int32),
          pltpu.VMEM((K1, CB), jnp.int32),
          pltpu.VMEM((CB, d), jnp.float32),
          pltpu.VMEM((ZR, d), jnp.float32),
          pltpu.VMEM_SHARED((NP, d), jnp.float32),
      ),
  )


RB = 1000
_G = N_NODES // RB


def _rows(w):
  return pl.BlockSpec((RB, w), lambda i: (i, 0))


def _full(a, b):
  return pl.BlockSpec((a, b), lambda i: (0, 0))


def _dot(a, w):
  return jnp.dot(a, w, precision=lax.Precision.HIGHEST,
                 preferred_element_type=jnp.float32)


def _m0_body(x, do0, do1, di0, di1, xs1, ns, nd):
  nsv = lax.rsqrt(jnp.maximum(do0[...] + do1[...], 1.0))
  ndv = lax.rsqrt(jnp.maximum(di0[...] + di1[...], 1.0))
  ns[...] = nsv
  nd[...] = ndv
  xs1[...] = x[...] * nsv


def _m1_body(p0, p1, nd, ns, w1, b1, w2, xs2):
  a = (p0[...] + p1[...]) * nd[...]
  h = jnp.maximum(_dot(a, w1[...]) + b1[...], 0.0)
  xs2[...] = _dot(h * ns[...], w2[...])


def _m2_body(p0, p1, nd, ns, b2, w3, xs3):
  h = jnp.maximum((p0[...] + p1[...]) * nd[...] + b2[...], 0.0)
  xs3[...] = _dot(h * ns[...], w3[...])


def _m3_body(p0, p1, nd, ns, b3, xs4):
  xs4[...] = jnp.maximum((p0[...] + p1[...]) * nd[...] + b3[...], 0.0) * ns[...]


def _m4_body(p0, p1, nd, w4, b4, o):
  a = (p0[...] + p1[...]) * nd[...]
  o[...] = jnp.maximum(_dot(a, w4[...]) + b4[...], 0.0)


def _tc_call(body, in_specs, out_shapes, out_specs):
  return pl.pallas_call(
      body,
      grid=(_G,),
      in_specs=in_specs,
      out_specs=out_specs,
      out_shape=out_shapes,
  )


def kernel(x, edge_index, W1, b1, W2, b2, W3, b3, W4, b4):
  src = edge_index[0].astype(jnp.int32)
  dst = edge_index[1].astype(jnp.int32)
  pad = EP - NE
  pidx = jnp.arange(pad, dtype=jnp.int32)
  pad_src = pidx % N_NODES
  pad_dum = N_NODES + pidx % (NP - N_NODES)
  src2 = jnp.concatenate([src, pad_src]).reshape(TCH, CB)
  srcd2 = jnp.concatenate([src, pad_dum]).reshape(TCH, CB)
  dst2 = jnp.concatenate([dst, pad_dum]).reshape(TCH, CB)

  dego_p, degi_p = _deg_kernel(srcd2, dst2)
  do0 = dego_p[0, :N_NODES, None]
  do1 = dego_p[1, :N_NODES, None]
  di0 = degi_p[0, :N_NODES, None]
  di1 = degi_p[1, :N_NODES, None]

  xs1, ns, nd = _tc_call(
      _m0_body,
      [_rows(128), _rows(1), _rows(1), _rows(1), _rows(1)],
      (jax.ShapeDtypeStruct((N_NODES, 128), jnp.float32),
       jax.ShapeDtypeStruct((N_NODES, 1), jnp.float32),
       jax.ShapeDtypeStruct((N_NODES, 1), jnp.float32)),
      (_rows(128), _rows(1), _rows(1)),
  )(x, do0, do1, di0, di1)

  a1 = _agg_kernel(128)(xs1, src2, dst2)
  xs2 = _tc_call(
      _m1_body,
      [_rows(128), _rows(128), _rows(1), _rows(1),
       _full(128, 512), _full(1, 512), _full(512, 64)],
      jax.ShapeDtypeStruct((N_NODES, 64), jnp.float32),
      _rows(64),
  )(a1[0, :N_NODES], a1[1, :N_NODES], nd, ns, W1, b1[None, :], W2)

  a2 = _agg_kernel(64)(xs2, src2, dst2)
  xs3 = _tc_call(
      _m2_body,
      [_rows(64), _rows(64), _rows(1), _rows(1), _full(1, 64), _full(64, 16)],
      jax.ShapeDtypeStruct((N_NODES, 16), jnp.float32),
      _rows(16),
  )(a2[0, :N_NODES], a2[1, :N_NODES], nd, ns, b2[None, :], W3)

  a3 = _agg_kernel(16)(xs3, src2, dst2)
  xs4 = _tc_call(
      _m3_body,
      [_rows(16), _rows(16), _rows(1), _rows(1), _full(1, 16)],
      jax.ShapeDtypeStruct((N_NODES, 16), jnp.float32),
      _rows(16),
  )(a3[0, :N_NODES], a3[1, :N_NODES], nd, ns, b3[None, :])

  a4 = _agg_kernel(16)(xs4, src2, dst2)
  out = _tc_call(
      _m4_body,
      [_rows(16), _rows(16), _rows(1), _full(16, 128), _full(1, 128)],
      jax.ShapeDtypeStruct((N_NODES, 128), jnp.float32),
      _rows(128),
  )(a4[0, :N_NODES], a4[1, :N_NODES], nd, W4, b4[None, :])
  return out


# speedup vs baseline: 2.1112x; 1.0992x over previous
import functools

import jax
import jax.numpy as jnp
from jax import lax
from jax.experimental import pallas as pl
from jax.experimental.pallas import tpu as pltpu
from jax.experimental.pallas import tpu_sc as plsc

N_NODES = 10000
NP = 10240
NE = 320000
NC = 2
NS = 16
NW = NC * NS
CB = 128
TCH = (NE + CB - 1) // CB + 60
K0 = 80
K1 = 80
CH = TCH // NW
EP = TCH * CB
RPT = NP // NS
ZR = 32

assert NS * (K0 + K1) == TCH

_MESH = plsc.VectorSubcoreMesh(core_axis_name="c", subcore_axis_name="s")


def _fill_const(ref, n16, val):
  v = jnp.full((16,), val, jnp.float32)

  @pl.loop(0, n16)
  def _(t):
    if len(ref.shape) == 1:
      ref[pl.ds(t * 16, 16)] = v
    else:
      w16 = ref.shape[1] // 16
      i = t // w16
      j = t % w16
      ref[i, pl.ds(j * 16, 16)] = v


def _deg_body(srcd_hbm, dstd_hbm, dego_hbm, degi_hbm,
              srcv, dstv, ones_v, zb, dego, degi):
  c = lax.axis_index("c")
  s = lax.axis_index("s")
  wid = c * NS + s
  row0 = s * RPT

  _fill_const(ones_v, CB // 16, 1.0)
  _fill_const(zb, RPT // 16, 0.0)
  pltpu.sync_copy(zb, dego.at[pl.ds(row0, RPT)])
  pltpu.sync_copy(zb, degi.at[pl.ds(row0, RPT)])
  pltpu.sync_copy(srcd_hbm.at[pl.ds(wid * CH, CH)], srcv)
  pltpu.sync_copy(dstd_hbm.at[pl.ds(wid * CH, CH)], dstv)
  plsc.subcore_barrier()

  @pl.loop(0, CH)
  def _(k):
    pltpu.sync_copy(ones_v, dego.at[srcv.at[k]], add=True)
    pltpu.sync_copy(ones_v, degi.at[dstv.at[k]], add=True)

  plsc.subcore_barrier()
  pltpu.sync_copy(dego.at[pl.ds(row0, RPT)], dego_hbm.at[c, pl.ds(row0, RPT)])
  pltpu.sync_copy(degi.at[pl.ds(row0, RPT)], degi_hbm.at[c, pl.ds(row0, RPT)])


_deg_kernel = pl.kernel(
    _deg_body,
    out_type=(
        jax.ShapeDtypeStruct((NC, NP), jnp.float32),
        jax.ShapeDtypeStruct((NC, NP), jnp.float32),
    ),
    mesh=_MESH,
    scratch_types=(
        pltpu.VMEM((CH, CB), jnp.int32),
        pltpu.VMEM((CH, CB), jnp.int32),
        pltpu.VMEM((CB,), jnp.float32),
        pltpu.VMEM((RPT,), jnp.float32),
        pltpu.VMEM_SHARED((NP,), jnp.float32),
        pltpu.VMEM_SHARED((NP,), jnp.float32),
    ),
)


def _agg_body(xs_hbm, src_hbm, dst_hbm, out_hbm, srcv, dstv, rows0, zb, acc):
  c = lax.axis_index("c")
  s = lax.axis_index("s")
  row0 = s * RPT
  d = xs_hbm.shape[1]

  _fill_const(zb, ZR * d // 16, 0.0)

  @pl.loop(0, RPT // ZR)
  def _(j):
    pltpu.sync_copy(zb, acc.at[pl.ds(row0 + j * ZR, ZR)])

  plsc.subcore_barrier()

  def run(base, nk):
    pltpu.sync_copy(src_hbm.at[pl.ds(base, nk)], srcv.at[pl.ds(0, nk)])
    pltpu.sync_copy(dst_hbm.at[pl.ds(base, nk)], dstv.at[pl.ds(0, nk)])

    @pl.loop(0, nk)
    def _(k):
      pltpu.sync_copy(xs_hbm.at[srcv.at[k]], rows0)
      pltpu.sync_copy(rows0, acc.at[dstv.at[k]], add=True)

  @pl.when(c == 0)
  def _():
    run(s * K0, K0)

  @pl.when(c == 1)
  def _():
    run(NS * K0 + s * K1, K1)

  plsc.subcore_barrier()
  pltpu.sync_copy(acc.at[pl.ds(row0, RPT)], out_hbm.at[c, pl.ds(row0, RPT)])


@functools.cache
def _agg_kernel(d):
  return pl.kernel(
      _agg_body,
      out_type=jax.ShapeDtypeStruct((NC, NP, d), jnp.float32),
      mesh=_MESH,
      compiler_params=pltpu.CompilerParams(use_tc_tiling_on_sc=False),
      scratch_types=(
          pltpu.VMEM((K1, CB), jnp.int32),
          pltpu.VMEM((K1, CB), jnp.int32),
          pltpu.VMEM((CB, d), jnp.float32),
          pltpu.VMEM((ZR, d), jnp.float32),
          pltpu.VMEM_SHARED((NP, d), jnp.float32),
      ),
  )


RB = 1000
_G = N_NODES // RB


def _rows(w):
  return pl.BlockSpec((RB, w), lambda i: (i, 0))


def _full(a, b):
  return pl.BlockSpec((a, b), lambda i: (0, 0))


def _part(c, w):
  return pl.BlockSpec((1, RB, w), lambda i, c=c: (c, i, 0))


def _dot(a, w):
  return jnp.dot(a, w, precision=lax.Precision.HIGHEST,
                 preferred_element_type=jnp.float32)


def _bdot(a, w):
  return jnp.dot(a.astype(jnp.bfloat16), w.astype(jnp.bfloat16),
                 preferred_element_type=jnp.float32)


def _m0_body(x, do0, do1, di0, di1, xs1, ns, nd):
  nsv = lax.rsqrt(jnp.maximum(do0[...] + do1[...], 1.0))
  ndv = lax.rsqrt(jnp.maximum(di0[...] + di1[...], 1.0))
  ns[...] = nsv
  nd[...] = ndv
  xs1[...] = x[...] * nsv


def _m1_body(p0, p1, nd, ns, w1, b1, w2, xs2):
  a = (p0[0] + p1[0]) * nd[...]
  h = jnp.maximum(_bdot(a, w1[...]) + b1[...], 0.0)
  xs2[...] = _bdot(h * ns[...], w2[...])


def _m2_body(p0, p1, nd, ns, b2, w3, xs3):
  h = jnp.maximum((p0[0] + p1[0]) * nd[...] + b2[...], 0.0)
  xs3[...] = _dot(h * ns[...], w3[...])


def _m3_body(p0, p1, nd, ns, b3, xs4):
  xs4[...] = jnp.maximum((p0[0] + p1[0]) * nd[...] + b3[...], 0.0) * ns[...]


def _m4_body(p0, p1, nd, w4, b4, o):
  a = (p0[0] + p1[0]) * nd[...]
  o[...] = jnp.maximum(_dot(a, w4[...]) + b4[...], 0.0)


def _tc_call(body, in_specs, out_shapes, out_specs):
  return pl.pallas_call(
      body,
      grid=(_G,),
      in_specs=in_specs,
      out_specs=out_specs,
      out_shape=out_shapes,
  )


def kernel(x, edge_index, W1, b1, W2, b2, W3, b3, W4, b4):
  src = edge_index[0].astype(jnp.int32)
  dst = edge_index[1].astype(jnp.int32)
  pad = EP - NE
  pidx = jnp.arange(pad, dtype=jnp.int32)
  pad_src = pidx % N_NODES
  pad_dum = N_NODES + pidx % (NP - N_NODES)
  src2 = jnp.concatenate([src, pad_src]).reshape(TCH, CB)
  srcd2 = jnp.concatenate([src, pad_dum]).reshape(TCH, CB)
  dst2 = jnp.concatenate([dst, pad_dum]).reshape(TCH, CB)

  dego_p, degi_p = _deg_kernel(srcd2, dst2)
  do0 = dego_p[0, :N_NODES, None]
  do1 = dego_p[1, :N_NODES, None]
  di0 = degi_p[0, :N_NODES, None]
  di1 = degi_p[1, :N_NODES, None]

  xs1, ns, nd = _tc_call(
      _m0_body,
      [_rows(128), _rows(1), _rows(1), _rows(1), _rows(1)],
      (jax.ShapeDtypeStruct((N_NODES, 128), jnp.float32),
       jax.ShapeDtypeStruct((N_NODES, 1), jnp.float32),
       jax.ShapeDtypeStruct((N_NODES, 1), jnp.float32)),
      (_rows(128), _rows(1), _rows(1)),
  )(x, do0, do1, di0, di1)

  a1 = _agg_kernel(128)(xs1, src2, dst2)
  xs2 = _tc_call(
      _m1_body,
      [_part(0, 128), _part(1, 128), _rows(1), _rows(1),
       _full(128, 512), _full(1, 512), _full(512, 64)],
      jax.ShapeDtypeStruct((N_NODES, 64), jnp.float32),
      _rows(64),
  )(a1, a1, nd, ns, W1, b1[None, :], W2)

  a2 = _agg_kernel(64)(xs2, src2, dst2)
  xs3 = _tc_call(
      _m2_body,
      [_part(0, 64), _part(1, 64), _rows(1), _rows(1), _full(1, 64), _full(64, 16)],
      jax.ShapeDtypeStruct((N_NODES, 16), jnp.float32),
      _rows(16),
  )(a2, a2, nd, ns, b2[None, :], W3)

  a3 = _agg_kernel(16)(xs3, src2, dst2)
  xs4 = _tc_call(
      _m3_body,
      [_part(0, 16), _part(1, 16), _rows(1), _rows(1), _full(1, 16)],
      jax.ShapeDtypeStruct((N_NODES, 16), jnp.float32),
      _rows(16),
  )(a3, a3, nd, ns, b3[None, :])

  a4 = _agg_kernel(16)(xs4, src2, dst2)
  out = _tc_call(
      _m4_body,
      [_part(0, 16), _part(1, 16), _rows(1), _full(16, 128), _full(1, 128)],
      jax.ShapeDtypeStruct((N_NODES, 128), jnp.float32),
      _rows(128),
  )(a4, a4, nd, W4, b4[None, :])
  return out
